# baseline (speedup 1.0000x reference)
import functools

import jax
import jax.numpy as jnp
from jax import lax
from jax.experimental import pallas as pl
from jax.experimental.pallas import tpu as pltpu
from jax.experimental.pallas import tpu_sc as plsc

_NUM_CODES = 1024
_DIM = 64


def _vq_tc_body(z_ref, cb_ref, idxf_ref, loss_ref):
    b = pl.program_id(0)
    p = z_ref.shape[2]
    zb = z_ref[0]
    cb = cb_ref[...]
    s = lax.dot_general(
        cb, zb, (((1,), (0,)), ((), ())),
        preferred_element_type=jnp.float32,
        precision=lax.Precision.DEFAULT,
    )
    z2 = jnp.sum(zb * zb, axis=0, keepdims=True)
    c2 = jnp.sum(cb * cb, axis=1, keepdims=True)
    d = (z2 + c2) - 2.0 * s
    min_d = jnp.min(d, axis=0, keepdims=True)
    cidx = lax.broadcasted_iota(jnp.int32, d.shape, 0)
    idx = jnp.min(
        jnp.where(d == min_d, cidx, jnp.int32(2**30)), axis=0, keepdims=True
    )
    idxf_ref[...] = idx.reshape(p)
    @pl.when(b == 0)
    def _():
        loss_ref[...] = jnp.zeros_like(loss_ref)
    loss_ref[...] += (jnp.sum(min_d) * (1.25 / 1048576.0)).reshape(1, 1)


def _vq_tc(z3, codebook, b0, nb):
    B, D, P = z3.shape
    return pl.pallas_call(
        _vq_tc_body,
        grid=(nb,),
        in_specs=[
            pl.BlockSpec((1, D, P), lambda b: (b + b0, 0, 0)),
            pl.BlockSpec((_NUM_CODES, D), lambda b: (0, 0)),
        ],
        out_specs=[
            pl.BlockSpec((P,), lambda b: (b,)),
            pl.BlockSpec((1, 1), lambda b: (0, 0)),
        ],
        out_shape=[
            jax.ShapeDtypeStruct((nb * P,), jnp.int32),
            jax.ShapeDtypeStruct((1, 1), jnp.float32),
        ],
    )(z3, codebook)


def _sc_gather(table_pad, idx_flat):
    info = plsc.get_sparse_core_info()
    nc, ns = info.num_cores, info.num_subcores
    nw = nc * ns
    n = idx_flat.shape[0]
    dpad = table_pad.shape[1]
    b_per_w = n // nw
    mesh = plsc.VectorSubcoreMesh(core_axis_name="c", subcore_axis_name="s")

    @functools.partial(
        pl.kernel,
        mesh=mesh,
        out_type=jax.ShapeDtypeStruct((n, dpad), jnp.float32),
        scratch_types=[
            pltpu.VMEM((b_per_w,), jnp.int32),
            pltpu.VMEM((b_per_w, dpad), jnp.float32),
            pltpu.SemaphoreType.DMA,
        ],
    )
    def k(table_hbm, idx_hbm, out_hbm, idx_v, rows_v, sem):
        wid = lax.axis_index("s") * nc + lax.axis_index("c")
        base = wid * b_per_w
        pltpu.sync_copy(idx_hbm.at[pl.ds(base, b_per_w)], idx_v)
        pltpu.async_copy(table_hbm.at[idx_v], rows_v, sem).wait()
        pltpu.sync_copy(rows_v, out_hbm.at[pl.ds(base, b_per_w)])

    return k(table_pad, idx_flat)


def kernel(z, codebook):
    B, D, H, W = z.shape
    P = H * W
    nbh = B // 2
    z3 = z.reshape(B, D, P)
    table_pad = jnp.pad(codebook, ((0, 0), (0, 128 - D)))
    idxf_a, loss_a = _vq_tc(z3, codebook, 0, nbh)
    zq_a = _sc_gather(table_pad, idxf_a)
    idxf_b, loss_b = _vq_tc(z3, codebook, nbh, nbh)
    zq_b = _sc_gather(table_pad, idxf_b)
    z_q_a = zq_a[:, :D].reshape(nbh, H, W, D).transpose(0, 3, 1, 2)
    z_q_b = zq_b[:, :D].reshape(nbh, H, W, D).transpose(0, 3, 1, 2)
    z_q = jnp.concatenate([z_q_a, z_q_b], axis=0)
    indices = jnp.concatenate([idxf_a, idxf_b]).reshape(B, H, W)
    loss = loss_a[0, 0] + loss_b[0, 0]
    return (z_q, loss, indices)


# speedup vs baseline: 1.1107x; 1.1107x over previous
import functools

import jax
import jax.numpy as jnp
from jax import lax
from jax.experimental import pallas as pl
from jax.experimental.pallas import tpu as pltpu
from jax.experimental.pallas import tpu_sc as plsc

_NUM_CODES = 1024
_DIM = 64


def _vq_tc_body(z_ref, cb_ref, idxf_ref, loss_ref):
    b = pl.program_id(0)
    p = z_ref.shape[2]
    zb = z_ref[0]
    cb = cb_ref[...]
    s = lax.dot_general(
        cb, zb, (((1,), (0,)), ((), ())),
        preferred_element_type=jnp.float32,
        precision=lax.Precision.DEFAULT,
    )
    z2 = jnp.sum(zb * zb, axis=0, keepdims=True)
    c2 = jnp.sum(cb * cb, axis=1, keepdims=True)
    d = (z2 + c2) - 2.0 * s
    min_d = jnp.min(d, axis=0, keepdims=True)
    cidx = lax.broadcasted_iota(jnp.int32, d.shape, 0)
    idx = jnp.min(
        jnp.where(d == min_d, cidx, jnp.int32(2**30)), axis=0, keepdims=True
    )
    idxf_ref[...] = idx.reshape(p)
    @pl.when(b == 0)
    def _():
        loss_ref[...] = jnp.zeros_like(loss_ref)
    loss_ref[...] += (jnp.sum(min_d) * (1.25 / 1048576.0)).reshape(1, 1)


def _vq_tc(z3, codebook):
    B, D, P = z3.shape
    return pl.pallas_call(
        _vq_tc_body,
        grid=(B,),
        in_specs=[
            pl.BlockSpec((1, D, P), lambda b: (b, 0, 0)),
            pl.BlockSpec((_NUM_CODES, D), lambda b: (0, 0)),
        ],
        out_specs=[
            pl.BlockSpec((P,), lambda b: (b,)),
            pl.BlockSpec((1, 1), lambda b: (0, 0)),
        ],
        out_shape=[
            jax.ShapeDtypeStruct((B * P,), jnp.int32),
            jax.ShapeDtypeStruct((1, 1), jnp.float32),
        ],
    )(z3, codebook)


def _sc_gather(table_pad, idx_flat):
    info = plsc.get_sparse_core_info()
    nc, ns = info.num_cores, info.num_subcores
    nw = nc * ns
    n = idx_flat.shape[0]
    dpad = table_pad.shape[1]
    b_per_w = n // nw
    mesh = plsc.VectorSubcoreMesh(core_axis_name="c", subcore_axis_name="s")

    @functools.partial(
        pl.kernel,
        mesh=mesh,
        out_type=jax.ShapeDtypeStruct((n, dpad), jnp.float32),
        scratch_types=[
            pltpu.VMEM((b_per_w,), jnp.int32),
            pltpu.VMEM((b_per_w, dpad), jnp.float32),
            pltpu.SemaphoreType.DMA,
        ],
    )
    def k(table_hbm, idx_hbm, out_hbm, idx_v, rows_v, sem):
        wid = lax.axis_index("s") * nc + lax.axis_index("c")
        base = wid * b_per_w
        pltpu.sync_copy(idx_hbm.at[pl.ds(base, b_per_w)], idx_v)
        pltpu.async_copy(table_hbm.at[idx_v], rows_v, sem).wait()
        pltpu.sync_copy(rows_v, out_hbm.at[pl.ds(base, b_per_w)])

    return k(table_pad, idx_flat)


def kernel(z, codebook):
    B, D, H, W = z.shape
    P = H * W
    z3 = z.reshape(B, D, P)
    idxf, loss_v = _vq_tc(z3, codebook)
    indices = idxf.reshape(B, H, W)
    table_pad = jnp.pad(codebook, ((0, 0), (0, 128 - D)))
    zq_flat = _sc_gather(table_pad, idxf)
    z_q = zq_flat[:, :D].reshape(B, H, W, D).transpose(0, 3, 1, 2)
    loss = loss_v[0, 0]
    return (z_q, loss, indices)


# speedup vs baseline: 1.1417x; 1.0279x over previous
import functools

import jax
import jax.numpy as jnp
from jax import lax
from jax.experimental import pallas as pl
from jax.experimental.pallas import tpu as pltpu
from jax.experimental.pallas import tpu_sc as plsc

_NUM_CODES = 1024
_DIM = 64


def _vq_tc_body(z_ref, cb_ref, idxf_ref, loss_ref):
    b = pl.program_id(0)
    nb = z_ref.shape[0]
    p = z_ref.shape[2]
    cb = cb_ref[...]
    c2 = jnp.sum(cb * cb, axis=1, keepdims=True)
    @pl.when(b == 0)
    def _():
        loss_ref[...] = jnp.zeros_like(loss_ref)
    for i in range(nb):
        zb = z_ref[i]
        s = lax.dot_general(
            cb, zb, (((1,), (0,)), ((), ())),
            preferred_element_type=jnp.float32,
            precision=lax.Precision.DEFAULT,
        )
        z2 = jnp.sum(zb * zb, axis=0, keepdims=True)
        d = (z2 + c2) - 2.0 * s
        min_d = jnp.min(d, axis=0, keepdims=True)
        cidx = lax.broadcasted_iota(jnp.int32, d.shape, 0)
        idx = jnp.min(
            jnp.where(d == min_d, cidx, jnp.int32(2**30)), axis=0,
            keepdims=True,
        )
        idxf_ref[pl.ds(i * p, p)] = idx.reshape(p)
        loss_ref[...] += (jnp.sum(min_d) * (1.25 / 1048576.0)).reshape(1, 1)


def _vq_tc(z3, codebook):
    B, D, P = z3.shape
    nb = 2
    return pl.pallas_call(
        _vq_tc_body,
        grid=(B // nb,),
        in_specs=[
            pl.BlockSpec((nb, D, P), lambda b: (b, 0, 0)),
            pl.BlockSpec((_NUM_CODES, D), lambda b: (0, 0)),
        ],
        out_specs=[
            pl.BlockSpec((nb * P,), lambda b: (b,)),
            pl.BlockSpec((1, 1), lambda b: (0, 0)),
        ],
        out_shape=[
            jax.ShapeDtypeStruct((B * P,), jnp.int32),
            jax.ShapeDtypeStruct((1, 1), jnp.float32),
        ],
    )(z3, codebook)


def _sc_gather(table_pad, idx_flat):
    info = plsc.get_sparse_core_info()
    nc, ns = info.num_cores, info.num_subcores
    nw = nc * ns
    n = idx_flat.shape[0]
    dpad = table_pad.shape[1]
    b_per_w = n // nw
    mesh = plsc.VectorSubcoreMesh(core_axis_name="c", subcore_axis_name="s")

    @functools.partial(
        pl.kernel,
        mesh=mesh,
        out_type=jax.ShapeDtypeStruct((n, dpad), jnp.float32),
        scratch_types=[
            pltpu.VMEM((b_per_w,), jnp.int32),
            pltpu.VMEM((b_per_w, dpad), jnp.float32),
            pltpu.SemaphoreType.DMA,
        ],
    )
    def k(table_hbm, idx_hbm, out_hbm, idx_v, rows_v, sem):
        wid = lax.axis_index("s") * nc + lax.axis_index("c")
        base = wid * b_per_w
        pltpu.sync_copy(idx_hbm.at[pl.ds(base, b_per_w)], idx_v)
        pltpu.async_copy(table_hbm.at[idx_v], rows_v, sem).wait()
        pltpu.sync_copy(rows_v, out_hbm.at[pl.ds(base, b_per_w)])

    return k(table_pad, idx_flat)


def kernel(z, codebook):
    B, D, H, W = z.shape
    P = H * W
    z3 = z.reshape(B, D, P)
    idxf, loss_v = _vq_tc(z3, codebook)
    indices = idxf.reshape(B, H, W)
    table_pad = jnp.pad(codebook, ((0, 0), (0, 128 - D)))
    zq_flat = _sc_gather(table_pad, idxf)
    z_q = zq_flat[:, :D].reshape(B, H, W, D).transpose(0, 3, 1, 2)
    loss = loss_v[0, 0]
    return (z_q, loss, indices)


# speedup vs baseline: 1.1701x; 1.0249x over previous
import functools

import jax
import jax.numpy as jnp
from jax import lax
from jax.experimental import pallas as pl
from jax.experimental.pallas import tpu as pltpu
from jax.experimental.pallas import tpu_sc as plsc

_NUM_CODES = 1024
_DIM = 64


def _vq_tc_body(z_ref, cb_ref, idxf_ref, loss_ref):
    b = pl.program_id(0)
    nb = z_ref.shape[0]
    p = z_ref.shape[2]
    cb = cb_ref[...]
    c2 = jnp.sum(cb * cb, axis=1, keepdims=True)
    @pl.when(b == 0)
    def _():
        loss_ref[...] = jnp.zeros_like(loss_ref)
    for i in range(nb):
        zb = z_ref[i]
        s = lax.dot_general(
            cb, zb, (((1,), (0,)), ((), ())),
            preferred_element_type=jnp.float32,
            precision=lax.Precision.DEFAULT,
        )
        z2 = jnp.sum(zb * zb, axis=0, keepdims=True)
        d = (z2 + c2) - 2.0 * s
        min_d = jnp.min(d, axis=0, keepdims=True)
        cidx = lax.broadcasted_iota(jnp.int32, d.shape, 0)
        idx = jnp.min(
            jnp.where(d == min_d, cidx, jnp.int32(2**30)), axis=0,
            keepdims=True,
        )
        idxf_ref[pl.ds(i * p, p)] = idx.reshape(p)
        loss_ref[...] += (jnp.sum(min_d) * (1.25 / 1048576.0)).reshape(1, 1)


def _vq_tc(z3, codebook):
    B, D, P = z3.shape
    nb = 4
    return pl.pallas_call(
        _vq_tc_body,
        grid=(B // nb,),
        in_specs=[
            pl.BlockSpec((nb, D, P), lambda b: (b, 0, 0)),
            pl.BlockSpec((_NUM_CODES, D), lambda b: (0, 0)),
        ],
        out_specs=[
            pl.BlockSpec((nb * P,), lambda b: (b,)),
            pl.BlockSpec((1, 1), lambda b: (0, 0)),
        ],
        out_shape=[
            jax.ShapeDtypeStruct((B * P,), jnp.int32),
            jax.ShapeDtypeStruct((1, 1), jnp.float32),
        ],
    )(z3, codebook)


def _sc_gather(table_pad, idx_flat):
    info = plsc.get_sparse_core_info()
    nc, ns = info.num_cores, info.num_subcores
    nw = nc * ns
    n = idx_flat.shape[0]
    dpad = table_pad.shape[1]
    b_per_w = n // nw
    mesh = plsc.VectorSubcoreMesh(core_axis_name="c", subcore_axis_name="s")

    @functools.partial(
        pl.kernel,
        mesh=mesh,
        out_type=jax.ShapeDtypeStruct((n, dpad), jnp.float32),
        scratch_types=[
            pltpu.VMEM((b_per_w,), jnp.int32),
            pltpu.VMEM((b_per_w, dpad), jnp.float32),
            pltpu.SemaphoreType.DMA,
        ],
    )
    def k(table_hbm, idx_hbm, out_hbm, idx_v, rows_v, sem):
        wid = lax.axis_index("s") * nc + lax.axis_index("c")
        base = wid * b_per_w
        pltpu.sync_copy(idx_hbm.at[pl.ds(base, b_per_w)], idx_v)
        pltpu.async_copy(table_hbm.at[idx_v], rows_v, sem).wait()
        pltpu.sync_copy(rows_v, out_hbm.at[pl.ds(base, b_per_w)])

    return k(table_pad, idx_flat)


def kernel(z, codebook):
    B, D, H, W = z.shape
    P = H * W
    z3 = z.reshape(B, D, P)
    idxf, loss_v = _vq_tc(z3, codebook)
    indices = idxf.reshape(B, H, W)
    table_pad = jnp.pad(codebook, ((0, 0), (0, 128 - D)))
    zq_flat = _sc_gather(table_pad, idxf)
    z_q = zq_flat[:, :D].reshape(B, H, W, D).transpose(0, 3, 1, 2)
    loss = loss_v[0, 0]
    return (z_q, loss, indices)
